# Initial kernel scaffold; baseline (speedup 1.0000x reference)
import jax
import jax.numpy as jnp
from jax.experimental import pallas as pl


def kernel(x, adj, nodes_u, nodes_v, W1, b1, W2, b2, fc_W, fc_b):
    raise NotImplementedError("write your pallas kernel here")


# speedup vs baseline: 2.7678x; 2.7678x over previous
import functools

import jax
import jax.numpy as jnp
from jax import lax
from jax.experimental import pallas as pl
from jax.experimental.pallas import tpu as pltpu
from jax.experimental.pallas import tpu_sc as plsc


def _linear(x, W):
    def body(x_ref, w_ref, o_ref):
        o_ref[...] = jnp.dot(x_ref[...], w_ref[...],
                             preferred_element_type=jnp.float32)

    return pl.pallas_call(
        body,
        out_shape=jax.ShapeDtypeStruct((x.shape[0], W.shape[1]), jnp.float32),
    )(x, W)


_BM = 200


def _gcn_layer_fused(adj, t, b, Wn):
    n = adj.shape[0]

    def body(adj_ref, t_ref, b_ref, w_ref, o_ref):
        acc = jnp.dot(adj_ref[...], t_ref[...],
                      preferred_element_type=jnp.float32)
        h = jnp.maximum(acc + b_ref[...], 0.0)
        o_ref[...] = jnp.dot(h, w_ref[...], preferred_element_type=jnp.float32)

    return pl.pallas_call(
        body,
        grid=(n // _BM,),
        in_specs=[
            pl.BlockSpec((_BM, n), lambda i: (i, 0)),
            pl.BlockSpec((n, t.shape[1]), lambda i: (0, 0)),
            pl.BlockSpec((1, b.shape[1]), lambda i: (0, 0)),
            pl.BlockSpec(Wn.shape, lambda i: (0, 0)),
        ],
        out_specs=pl.BlockSpec((_BM, Wn.shape[1]), lambda i: (i, 0)),
        out_shape=jax.ShapeDtypeStruct((n, Wn.shape[1]), jnp.float32),
    )(adj, t, b, Wn)


def _gcn_layer_final(adj, t, b, Wu, Wv, fcb):
    n = adj.shape[0]
    d = Wu.shape[1]

    def body(adj_ref, t_ref, b_ref, wu_ref, wv_ref, fcb_ref, pu_ref, pv_ref):
        acc = jnp.dot(adj_ref[...], t_ref[...],
                      preferred_element_type=jnp.float32)
        h = jnp.maximum(acc + b_ref[...], 0.0)
        pu_ref[...] = jnp.dot(h, wu_ref[...],
                              preferred_element_type=jnp.float32) + fcb_ref[...]
        pv_ref[...] = jnp.dot(h, wv_ref[...],
                              preferred_element_type=jnp.float32)

    return pl.pallas_call(
        body,
        grid=(n // _BM,),
        in_specs=[
            pl.BlockSpec((_BM, n), lambda i: (i, 0)),
            pl.BlockSpec((n, t.shape[1]), lambda i: (0, 0)),
            pl.BlockSpec((1, b.shape[1]), lambda i: (0, 0)),
            pl.BlockSpec(Wu.shape, lambda i: (0, 0)),
            pl.BlockSpec(Wv.shape, lambda i: (0, 0)),
            pl.BlockSpec((1, d), lambda i: (0, 0)),
        ],
        out_specs=[
            pl.BlockSpec((_BM, d), lambda i: (i, 0)),
            pl.BlockSpec((_BM, d), lambda i: (i, 0)),
        ],
        out_shape=[
            jax.ShapeDtypeStruct((n, d), jnp.float32),
            jax.ShapeDtypeStruct((n, d), jnp.float32),
        ],
    )(adj, t, b, Wu, Wv, fcb)


_CHUNK = 80


def _edge_scores_sc(pu, pv, nodes_u, nodes_v):
    e = nodes_u.shape[0]
    d = pu.shape[1]
    info = plsc.get_sparse_core_info()
    nc, ns = info.num_cores, info.num_subcores
    nw = nc * ns
    epw = e // nw
    nch = epw // _CHUNK

    mesh = plsc.VectorSubcoreMesh(core_axis_name="c", subcore_axis_name="s")

    @functools.partial(
        pl.kernel,
        mesh=mesh,
        out_type=jax.ShapeDtypeStruct((e, d), jnp.float32),
        scratch_types=[
            pltpu.VMEM((_CHUNK,), jnp.int32),
            pltpu.VMEM((_CHUNK,), jnp.int32),
            pltpu.VMEM((_CHUNK, d), jnp.float32),
            pltpu.VMEM((_CHUNK, d), jnp.float32),
            pltpu.SemaphoreType.DMA,
            pltpu.SemaphoreType.DMA,
        ],
    )
    def k(pu_hbm, pv_hbm, u_hbm, v_hbm, out_hbm, iu, iv, bu, bv, su, sv):
        wid = lax.axis_index("s") * nc + lax.axis_index("c")

        def chunk_body(g, carry):
            base = pl.multiple_of(wid * epw + g * _CHUNK, 8)
            pltpu.sync_copy(u_hbm.at[pl.ds(base, _CHUNK)], iu)
            pltpu.sync_copy(v_hbm.at[pl.ds(base, _CHUNK)], iv)
            cu = pltpu.async_copy(pu_hbm.at[iu], bu, su)
            cv = pltpu.async_copy(pv_hbm.at[iv], bv, sv)
            cu.wait()
            cv.wait()

            def add_row(r, c2):
                for j in range(d // 16):
                    sl = pl.ds(j * 16, 16)
                    bu[r, sl] = bu[r, sl] + bv[r, sl]
                return c2

            lax.fori_loop(0, _CHUNK, add_row, 0)
            pltpu.sync_copy(bu, out_hbm.at[pl.ds(base, _CHUNK)])
            return carry

        lax.fori_loop(0, nch, chunk_body, 0)

    return k(pu, pv, nodes_u, nodes_v)


def kernel(x, adj, nodes_u, nodes_v, W1, b1, W2, b2, fc_W, fc_b):
    d = fc_W.shape[1]
    t1 = _linear(x, W1)
    t2 = _gcn_layer_fused(adj, t1, b1.reshape(1, -1), W2)
    pu, pv = _gcn_layer_final(adj, t2, b2.reshape(1, -1),
                              fc_W[:d], fc_W[d:], fc_b.reshape(1, -1))
    return _edge_scores_sc(pu, pv, nodes_u, nodes_v)


# speedup vs baseline: 3.8579x; 1.3939x over previous
import functools

import jax
import jax.numpy as jnp
from jax import lax
from jax.experimental import pallas as pl
from jax.experimental.pallas import tpu as pltpu
from jax.experimental.pallas import tpu_sc as plsc


def _linear(x, W):
    def body(x_ref, w_ref, o_ref):
        o_ref[...] = jnp.dot(x_ref[...], w_ref[...],
                             preferred_element_type=jnp.float32)

    return pl.pallas_call(
        body,
        out_shape=jax.ShapeDtypeStruct((x.shape[0], W.shape[1]), jnp.float32),
    )(x, W)


_BM = 200


def _gcn_layer_fused(adj, t, b, Wn):
    n = adj.shape[0]

    def body(adj_ref, t_ref, b_ref, w_ref, o_ref):
        acc = jnp.dot(adj_ref[...], t_ref[...],
                      preferred_element_type=jnp.float32)
        h = jnp.maximum(acc + b_ref[...], 0.0)
        o_ref[...] = jnp.dot(h, w_ref[...], preferred_element_type=jnp.float32)

    return pl.pallas_call(
        body,
        grid=(n // _BM,),
        in_specs=[
            pl.BlockSpec((_BM, n), lambda i: (i, 0)),
            pl.BlockSpec((n, t.shape[1]), lambda i: (0, 0)),
            pl.BlockSpec((1, b.shape[1]), lambda i: (0, 0)),
            pl.BlockSpec(Wn.shape, lambda i: (0, 0)),
        ],
        out_specs=pl.BlockSpec((_BM, Wn.shape[1]), lambda i: (i, 0)),
        out_shape=jax.ShapeDtypeStruct((n, Wn.shape[1]), jnp.float32),
    )(adj, t, b, Wn)


def _gcn_layer_final(adj, t, b, Wu, Wv, fcb):
    n = adj.shape[0]
    d = Wu.shape[1]

    def body(adj_ref, t_ref, b_ref, wu_ref, wv_ref, fcb_ref, pu_ref, pv_ref):
        acc = jnp.dot(adj_ref[...], t_ref[...],
                      preferred_element_type=jnp.float32)
        h = jnp.maximum(acc + b_ref[...], 0.0)
        pu_ref[...] = jnp.dot(h, wu_ref[...],
                              preferred_element_type=jnp.float32) + fcb_ref[...]
        pv_ref[...] = jnp.dot(h, wv_ref[...],
                              preferred_element_type=jnp.float32)

    return pl.pallas_call(
        body,
        grid=(n // _BM,),
        in_specs=[
            pl.BlockSpec((_BM, n), lambda i: (i, 0)),
            pl.BlockSpec((n, t.shape[1]), lambda i: (0, 0)),
            pl.BlockSpec((1, b.shape[1]), lambda i: (0, 0)),
            pl.BlockSpec(Wu.shape, lambda i: (0, 0)),
            pl.BlockSpec(Wv.shape, lambda i: (0, 0)),
            pl.BlockSpec((1, d), lambda i: (0, 0)),
        ],
        out_specs=[
            pl.BlockSpec((_BM, d), lambda i: (i, 0)),
            pl.BlockSpec((_BM, d), lambda i: (i, 0)),
        ],
        out_shape=[
            jax.ShapeDtypeStruct((n, d), jnp.float32),
            jax.ShapeDtypeStruct((n, d), jnp.float32),
        ],
    )(adj, t, b, Wu, Wv, fcb)


_CHUNK = 80


def _edge_scores_sc(pu, pv, nodes_u, nodes_v):
    e = nodes_u.shape[0]
    d = pu.shape[1]
    info = plsc.get_sparse_core_info()
    nc, ns = info.num_cores, info.num_subcores
    nw = nc * ns
    epw = e // nw
    nch = epw // _CHUNK

    mesh = plsc.VectorSubcoreMesh(core_axis_name="c", subcore_axis_name="s")

    @functools.partial(
        pl.kernel,
        mesh=mesh,
        out_type=jax.ShapeDtypeStruct((e, d), jnp.float32),
        scratch_types=[
            pltpu.VMEM((_CHUNK,), jnp.int32),
            pltpu.VMEM((_CHUNK,), jnp.int32),
            pltpu.VMEM((_CHUNK,), jnp.int32),
            pltpu.VMEM((_CHUNK,), jnp.int32),
            pltpu.VMEM((_CHUNK, d), jnp.float32),
            pltpu.VMEM((_CHUNK, d), jnp.float32),
            pltpu.VMEM((_CHUNK, d), jnp.float32),
            pltpu.VMEM((_CHUNK, d), jnp.float32),
            pltpu.VMEM((_CHUNK, d), jnp.float32),
            pltpu.VMEM((_CHUNK, d), jnp.float32),
            pltpu.SemaphoreType.DMA,
            pltpu.SemaphoreType.DMA,
            pltpu.SemaphoreType.DMA,
            pltpu.SemaphoreType.DMA,
        ],
    )
    def k(pu_hbm, pv_hbm, u_hbm, v_hbm, out_hbm,
          iu0, iu1, iv0, iv1, bu0, bu1, bv0, bv1, bo0, bo1,
          sg0, sg1, so0, so1):
        wid = lax.axis_index("s") * nc + lax.axis_index("c")
        slots = ((iu0, iv0, bu0, bv0, bo0, sg0, so0),
                 (iu1, iv1, bu1, bv1, bo1, sg1, so1))

        def base_of(g):
            return pl.multiple_of(wid * epw + g * _CHUNK, 8)

        def issue(g, sl):
            iu, iv, bu, bv, _, sg, _ = sl
            base = base_of(g)
            pltpu.sync_copy(u_hbm.at[pl.ds(base, _CHUNK)], iu)
            pltpu.sync_copy(v_hbm.at[pl.ds(base, _CHUNK)], iv)
            pltpu.async_copy(pu_hbm.at[iu], bu, sg)
            pltpu.async_copy(pv_hbm.at[iv], bv, sg)

        for s in (0, 1):
            issue(s, slots[s])

        def body(i, carry):
            for s in (0, 1):
                g = 2 * i + s
                iu, iv, bu, bv, bo, sg, so = slots[s]

                @pl.when(g < nch)
                def _():
                    pltpu.make_async_copy(pu_hbm.at[iu], bu, sg).wait()
                    pltpu.make_async_copy(pv_hbm.at[iv], bv, sg).wait()

                    @pl.when(g >= 2)
                    def _():
                        pltpu.make_async_copy(
                            bo, out_hbm.at[pl.ds(0, _CHUNK)], so).wait()

                    def add_row(r, c2):
                        for j in range(d // 16):
                            sl_ = pl.ds(j * 16, 16)
                            bo[r, sl_] = bu[r, sl_] + bv[r, sl_]
                        return c2

                    lax.fori_loop(0, _CHUNK, add_row, 0)
                    pltpu.async_copy(bo, out_hbm.at[pl.ds(base_of(g), _CHUNK)],
                                     so)

                    @pl.when(g + 2 < nch)
                    def _():
                        issue(g + 2, slots[s])
            return carry

        lax.fori_loop(0, (nch + 1) // 2, body, 0)
        for s in (0, 1):
            bo, so = slots[s][4], slots[s][6]
            pltpu.make_async_copy(bo, out_hbm.at[pl.ds(0, _CHUNK)], so).wait()

    return k(pu, pv, nodes_u, nodes_v)


def kernel(x, adj, nodes_u, nodes_v, W1, b1, W2, b2, fc_W, fc_b):
    d = fc_W.shape[1]
    t1 = _linear(x, W1)
    t2 = _gcn_layer_fused(adj, t1, b1.reshape(1, -1), W2)
    pu, pv = _gcn_layer_final(adj, t2, b2.reshape(1, -1),
                              fc_W[:d], fc_W[d:], fc_b.reshape(1, -1))
    return _edge_scores_sc(pu, pv, nodes_u, nodes_v)
